# baseline (speedup 1.0000x reference)
import functools

import jax
import jax.numpy as jnp
from jax import lax
from jax.experimental import pallas as pl
from jax.experimental.pallas import tpu as pltpu
from jax.experimental.pallas import tpu_sc as plsc

N = 10000
E = 320000
H = 128
G = 64

NCORES = 2
NSUB = 16
NW = NCORES * NSUB
EPT = E // NW
CHUNK = 125
NCH = EPT // CHUNK
NPAD = 10240
NPT = NPAD // NSUB


def _sc_agg(x, edges4, zeros_tile):
    mesh = plsc.VectorSubcoreMesh(core_axis_name="c", subcore_axis_name="s")

    @functools.partial(
        pl.kernel,
        out_type=jax.ShapeDtypeStruct((NCORES, NPAD, H), jnp.float32),
        mesh=mesh,
        scratch_types=[
            pltpu.VMEM((NCH, CHUNK), jnp.int32),
            pltpu.VMEM((NCH, CHUNK), jnp.int32),
            pltpu.VMEM((CHUNK, H), jnp.float32),
            pltpu.VMEM_SHARED((NPAD, H), jnp.float32),
            pltpu.SemaphoreType.DMA,
        ],
    )
    def k(x_hbm, edges_hbm, zero_hbm, out_hbm, src_v, dst_v, rows_v, agg, sem0):
        c = lax.axis_index("c")
        s = lax.axis_index("s")
        wid = s * NCORES + c

        pltpu.sync_copy(edges_hbm.at[0, wid], src_v)
        pltpu.sync_copy(edges_hbm.at[1, wid], dst_v)
        pltpu.sync_copy(zero_hbm, agg.at[pl.ds(s * NPT, NPT)])
        plsc.subcore_barrier()


        def body(j, carry):
            pltpu.async_copy(x_hbm.at[src_v.at[j]], rows_v, sem0,
                            priority=1).wait()
            pltpu.sync_copy(rows_v, agg.at[dst_v.at[j]], add=True)
            return carry

        lax.fori_loop(0, NCH, body, 0, unroll=False)

        plsc.subcore_barrier()
        pltpu.sync_copy(agg.at[pl.ds(s * NPT, NPT)],
                        out_hbm.at[c, pl.ds(s * NPT, NPT)])

    return k(x, edges4, zeros_tile)


def _gin_bn(xp_ref, p_ref, wa_ref, ba_ref, wb_ref, bb_ref, g_ref, be_ref):
    h = xp_ref[:N, :] + p_ref[0, :N, :] + p_ref[1, :N, :]
    t = jnp.maximum(
        jnp.dot(h, wa_ref[...], preferred_element_type=jnp.float32) + ba_ref[...],
        0.0)
    u = jnp.dot(t, wb_ref[...], preferred_element_type=jnp.float32) + bb_ref[...]
    mu = jnp.mean(u, axis=0, keepdims=True)
    d = u - mu
    var = jnp.mean(d * d, axis=0, keepdims=True)
    return jnp.maximum(d * lax.rsqrt(var + 1e-5) * g_ref[...] + be_ref[...], 0.0)


def _tc_layer_body(xp_ref, p_ref, wa_ref, ba_ref, wb_ref, bb_ref,
                   g_ref, be_ref, o_ref):
    o_ref[:N, :] = _gin_bn(xp_ref, p_ref, wa_ref, ba_ref, wb_ref, bb_ref,
                           g_ref, be_ref)
    o_ref[N:, :] = jnp.zeros((NPAD - N, H), jnp.float32)


def _tc_layer(xp, parts, Wa, ba, Wb, bb, g, be):
    return pl.pallas_call(
        _tc_layer_body,
        out_shape=jax.ShapeDtypeStruct((NPAD, H), jnp.float32),
    )(xp, parts, Wa, ba.reshape(1, H), Wb, bb.reshape(1, H),
      g.reshape(1, H), be.reshape(1, H))


def _tc_last_body(xp_ref, p_ref, wa_ref, ba_ref, wb_ref, bb_ref, g_ref, be_ref,
                  b_ref, wf1_ref, bf1_ref, wf2_ref, bf2_ref, o_ref):
    h = _gin_bn(xp_ref, p_ref, wa_ref, ba_ref, wb_ref, bb_ref, g_ref, be_ref)
    b = b_ref[...]
    gids = lax.broadcasted_iota(jnp.int32, (G, N), 0)
    onehot = (gids == b).astype(jnp.float32)
    sums = jnp.dot(onehot, h, preferred_element_type=jnp.float32)
    cnt = jnp.sum(onehot, axis=1, keepdims=True)
    pooled = sums / jnp.maximum(cnt, 1.0)
    t = jnp.maximum(
        jnp.dot(pooled, wf1_ref[...], preferred_element_type=jnp.float32)
        + bf1_ref[...], 0.0)
    o = jnp.dot(t, wf2_ref[...], preferred_element_type=jnp.float32) + bf2_ref[...]
    o_ref[...] = jnp.maximum(o, 0.0)


def _tc_last(xp, parts, Wa, ba, Wb, bb, g, be, batch2d, Wf1, bf1, Wf2, bf2):
    return pl.pallas_call(
        _tc_last_body,
        out_shape=jax.ShapeDtypeStruct((G, 1), jnp.float32),
    )(xp, parts, Wa, ba.reshape(1, H), Wb, bb.reshape(1, H),
      g.reshape(1, H), be.reshape(1, H), batch2d,
      Wf1, bf1.reshape(1, H), Wf2, bf2.reshape(1, 1))


def kernel(x, edge_index, edge_attr, batch,
           W1a, b1a, W1b, b1b, W2a, b2a, W2b, b2b, W3a, b3a, W3b, b3b,
           g1, be1, g2, be2, g3, be3, Wf1, bf1, Wf2, bf2):
    edges4 = edge_index.reshape(2, NW, NCH, CHUNK)
    zeros_tile = jnp.zeros((NPT, H), jnp.float32)
    batch2d = batch.reshape(1, N)

    hp = jnp.pad(x, ((0, NPAD - N), (0, 0)))
    for (Wa, ba, Wb, bb, g, be) in (
            (W1a, b1a, W1b, b1b, g1, be1),
            (W2a, b2a, W2b, b2b, g2, be2)):
        parts = _sc_agg(hp, edges4, zeros_tile)
        hp = _tc_layer(hp, parts, Wa, ba, Wb, bb, g, be)

    parts = _sc_agg(hp, edges4, zeros_tile)
    return _tc_last(hp, parts, W3a, b3a, W3b, b3b, g3, be3,
                    batch2d, Wf1, bf1, Wf2, bf2)


# speedup vs baseline: 1.0146x; 1.0146x over previous
import functools

import jax
import jax.numpy as jnp
from jax import lax
from jax.experimental import pallas as pl
from jax.experimental.pallas import tpu as pltpu
from jax.experimental.pallas import tpu_sc as plsc

N = 10000
E = 320000
H = 128
G = 64

NCORES = 2
NSUB = 16
NW = NCORES * NSUB
EPT = E // NW
CHUNK = 125
NCH = EPT // CHUNK
NPAD = 10240
NPT = NPAD // NSUB


def _sc_agg(x, edges4, zeros_tile):
    mesh = plsc.VectorSubcoreMesh(core_axis_name="c", subcore_axis_name="s")

    @functools.partial(
        pl.kernel,
        out_type=jax.ShapeDtypeStruct((NCORES, NPAD, H), jnp.float32),
        mesh=mesh,
        scratch_types=[
            pltpu.VMEM((NCH, CHUNK), jnp.int32),
            pltpu.VMEM((NCH, CHUNK), jnp.int32),
            pltpu.VMEM((CHUNK, H), jnp.float32),
            pltpu.VMEM_SHARED((NPAD, H), jnp.float32),
            pltpu.SemaphoreType.DMA,
        ],
    )
    def k(x_hbm, edges_hbm, zero_hbm, out_hbm, src_v, dst_v, rows_v, agg, sem0):
        c = lax.axis_index("c")
        s = lax.axis_index("s")
        wid = s * NCORES + c

        pltpu.sync_copy(edges_hbm.at[0, wid], src_v)
        pltpu.sync_copy(edges_hbm.at[1, wid], dst_v)
        @pl.when(c == 0)
        def _():
            pltpu.sync_copy(x_hbm.at[pl.ds(s * NPT, NPT)],
                            agg.at[pl.ds(s * NPT, NPT)])

        @pl.when(c == 1)
        def _():
            pltpu.sync_copy(zero_hbm, agg.at[pl.ds(s * NPT, NPT)])
        plsc.subcore_barrier()


        def body(j, carry):
            pltpu.async_copy(x_hbm.at[src_v.at[j]], rows_v, sem0).wait()
            pltpu.sync_copy(rows_v, agg.at[dst_v.at[j]], add=True)
            return carry

        lax.fori_loop(0, NCH, body, 0, unroll=False)

        plsc.subcore_barrier()
        pltpu.sync_copy(agg.at[pl.ds(s * NPT, NPT)],
                        out_hbm.at[c, pl.ds(s * NPT, NPT)])

    return k(x, edges4, zeros_tile)


def _gin_bn(p_ref, wa_ref, ba_ref, wb_ref, bb_ref, g_ref, be_ref):
    h = p_ref[0, :N, :] + p_ref[1, :N, :]
    t = jnp.maximum(
        jnp.dot(h, wa_ref[...], preferred_element_type=jnp.float32) + ba_ref[...],
        0.0)
    u = jnp.dot(t, wb_ref[...], preferred_element_type=jnp.float32) + bb_ref[...]
    mu = jnp.mean(u, axis=0, keepdims=True)
    d = u - mu
    var = jnp.mean(d * d, axis=0, keepdims=True)
    return jnp.maximum(d * lax.rsqrt(var + 1e-5) * g_ref[...] + be_ref[...], 0.0)


def _tc_layer_body(p_ref, wa_ref, ba_ref, wb_ref, bb_ref,
                   g_ref, be_ref, o_ref):
    o_ref[:N, :] = _gin_bn(p_ref, wa_ref, ba_ref, wb_ref, bb_ref,
                           g_ref, be_ref)
    o_ref[N:, :] = jnp.zeros((NPAD - N, H), jnp.float32)


def _tc_layer(parts, Wa, ba, Wb, bb, g, be):
    return pl.pallas_call(
        _tc_layer_body,
        out_shape=jax.ShapeDtypeStruct((NPAD, H), jnp.float32),
    )(parts, Wa, ba.reshape(1, H), Wb, bb.reshape(1, H),
      g.reshape(1, H), be.reshape(1, H))


def _tc_last_body(p_ref, wa_ref, ba_ref, wb_ref, bb_ref, g_ref, be_ref,
                  b_ref, wf1_ref, bf1_ref, wf2_ref, bf2_ref, o_ref):
    h = _gin_bn(p_ref, wa_ref, ba_ref, wb_ref, bb_ref, g_ref, be_ref)
    b = b_ref[...]
    gids = lax.broadcasted_iota(jnp.int32, (G, N), 0)
    onehot = (gids == b).astype(jnp.float32)
    sums = jnp.dot(onehot, h, preferred_element_type=jnp.float32)
    cnt = jnp.sum(onehot, axis=1, keepdims=True)
    pooled = sums / jnp.maximum(cnt, 1.0)
    t = jnp.maximum(
        jnp.dot(pooled, wf1_ref[...], preferred_element_type=jnp.float32)
        + bf1_ref[...], 0.0)
    o = jnp.dot(t, wf2_ref[...], preferred_element_type=jnp.float32) + bf2_ref[...]
    o_ref[...] = jnp.maximum(o, 0.0)


def _tc_last(parts, Wa, ba, Wb, bb, g, be, batch2d, Wf1, bf1, Wf2, bf2):
    return pl.pallas_call(
        _tc_last_body,
        out_shape=jax.ShapeDtypeStruct((G, 1), jnp.float32),
    )(parts, Wa, ba.reshape(1, H), Wb, bb.reshape(1, H),
      g.reshape(1, H), be.reshape(1, H), batch2d,
      Wf1, bf1.reshape(1, H), Wf2, bf2.reshape(1, 1))


def kernel(x, edge_index, edge_attr, batch,
           W1a, b1a, W1b, b1b, W2a, b2a, W2b, b2b, W3a, b3a, W3b, b3b,
           g1, be1, g2, be2, g3, be3, Wf1, bf1, Wf2, bf2):
    edges4 = edge_index.reshape(2, NW, NCH, CHUNK)
    zeros_tile = jnp.zeros((NPT, H), jnp.float32)
    batch2d = batch.reshape(1, N)

    hp = jnp.pad(x, ((0, NPAD - N), (0, 0)))
    for (Wa, ba, Wb, bb, g, be) in (
            (W1a, b1a, W1b, b1b, g1, be1),
            (W2a, b2a, W2b, b2b, g2, be2)):
        parts = _sc_agg(hp, edges4, zeros_tile)
        hp = _tc_layer(parts, Wa, ba, Wb, bb, g, be)

    parts = _sc_agg(hp, edges4, zeros_tile)
    return _tc_last(parts, W3a, b3a, W3b, b3b, g3, be3,
                    batch2d, Wf1, bf1, Wf2, bf2)
